# baseline (speedup 1.0000x reference)
import functools

import jax
import jax.numpy as jnp
from jax import lax
from jax.experimental import pallas as pl
from jax.experimental.pallas import tpu as pltpu
from jax.experimental.pallas import tpu_sc as plsc

H, W = 304, 608
HW = H * W
NF = 32
NC, NS, L = 2, 16, 16
ROWS_PER_CHUNK = 4
CB = ROWS_PER_CHUNK * W
NCHUNK = H // ROWS_PER_CHUNK
NITER = NCHUNK // 2
GPR = W // L

_mesh = plsc.VectorSubcoreMesh(
    core_axis_name="c", subcore_axis_name="s", num_cores=NC, num_subcores=NS)

_SET = [
    pltpu.VMEM((4 * CB,), jnp.int32),
    pltpu.VMEM((4 * CB,), jnp.float32),
    pltpu.VMEM((4 * CB,), jnp.float32),
]


@functools.partial(
    pl.kernel,
    out_type=jax.ShapeDtypeStruct((NF * HW,), jnp.float32),
    mesh=_mesh,
    scratch_types=[
        pltpu.VMEM((16,), jnp.float32),
        pltpu.VMEM((W,), jnp.float32),
        pltpu.VMEM((H * L,), jnp.float32),
        pltpu.VMEM((H * L,), jnp.float32),
        *_SET,
        *_SET,
        pltpu.VMEM((CB,), jnp.float32),
        pltpu.VMEM((CB,), jnp.float32),
        pltpu.SemaphoreType.DMA,
        pltpu.SemaphoreType.DMA,
        pltpu.SemaphoreType.DMA,
        pltpu.SemaphoreType.DMA,
    ],
)
def _warp(im_hbm, consts_hbm, xg_hbm, rx_hbm, ry_hbm, out_hbm,
          cv, xgv, rxv, ryv,
          ix0, wt0, gt0, ix1, wt1, gt1,
          ob0, ob1, sem0, sem1, semw0, semw1):
    wid = lax.axis_index("s") * NC + lax.axis_index("c")
    pltpu.sync_copy(consts_hbm.at[wid], cv)
    pltpu.sync_copy(xg_hbm, xgv)
    pltpu.sync_copy(rx_hbm.at[wid], rxv)
    pltpu.sync_copy(ry_hbm.at[wid], ryv)
    cvv = cv[...]
    a00 = jnp.full((L,), cvv[0])
    a02 = jnp.full((L,), cvv[1])
    a10 = jnp.full((L,), cvv[2])
    a12 = jnp.full((L,), cvv[3])
    base = wid * HW

    def compute(c, ix, wt):
        def row_body(r, rc):
            i = c * ROWS_PER_CHUNK + r
            px = rxv[pl.ds(i * L, L)]
            py = ryv[pl.ds(i * L, L)]

            def grp_body(g, _):
                xv = xgv[pl.ds(g * L, L)]
                xs = (a00 * xv + px) + a02
                ys = (a10 * xv + py) + a12
                xp = (xs + 1.0) * (W / 2)
                yp = (ys + 1.0) * (H / 2)
                xq = jnp.clip(xp, -1e6, 1e6)
                yq = jnp.clip(yp, -1e6, 1e6)
                xt = xq.astype(jnp.int32)
                xtf = xt.astype(jnp.float32)
                neg_x = xq < xtf
                fxl = jnp.where(neg_x, xt - 1, xt)
                fxf = jnp.where(neg_x, xtf - 1.0, xtf)
                yt = yq.astype(jnp.int32)
                ytf = yt.astype(jnp.float32)
                neg_y = yq < ytf
                fyl = jnp.where(neg_y, yt - 1, yt)
                fyf = jnp.where(neg_y, ytf - 1.0, ytf)
                in_x = (xp >= 0.0) & (xp < W - 1.0)
                in_y = (yp >= 0.0) & (yp < H - 1.0)
                zero = jnp.zeros((L,), jnp.float32)
                hl = jnp.where(in_x, fxf + 1.0 - xp, zero)
                hr = jnp.where(in_x, xp - fxf, zero)
                vt = jnp.where(in_y, fyf + 1.0 - yp, zero)
                vb = jnp.where(in_y, yp - fyf, zero)
                x0 = jnp.clip(fxl, 0, W - 2)
                y0 = jnp.clip(fyl, 0, H - 2)
                idx = base + y0 * W + x0
                o = (r * GPR + g) * L
                ix[pl.ds(o, L)] = idx
                ix[pl.ds(CB + o, L)] = idx + W
                ix[pl.ds(2 * CB + o, L)] = idx + 1
                ix[pl.ds(3 * CB + o, L)] = idx + (W + 1)
                wt[pl.ds(o, L)] = hl * vt
                wt[pl.ds(CB + o, L)] = hl * vb
                wt[pl.ds(2 * CB + o, L)] = hr * vt
                wt[pl.ds(3 * CB + o, L)] = hr * vb
                return 0

            lax.fori_loop(0, GPR, grp_body, 0)
            return rc

        lax.fori_loop(0, ROWS_PER_CHUNK, row_body, 0)

    def combine_write(c, wt, gt, ob, semw, first):
        @pl.when(jnp.logical_not(first))
        def _():
            pltpu.make_async_copy(im_hbm.at[pl.ds(0, CB)], ob, semw).wait()

        def comb(t, _):
            sl = pl.ds(t * L, L)
            ob[sl] = (wt[sl] * gt[sl]
                      + wt[pl.ds(CB + t * L, L)] * gt[pl.ds(CB + t * L, L)]
                      + wt[pl.ds(2 * CB + t * L, L)] * gt[pl.ds(2 * CB + t * L, L)]
                      + wt[pl.ds(3 * CB + t * L, L)] * gt[pl.ds(3 * CB + t * L, L)])
            return 0

        lax.fori_loop(0, CB // L, comb, 0)
        pltpu.async_copy(ob, out_hbm.at[pl.ds(base + c * CB, CB)], semw)

    compute(0, ix0, wt0)
    pltpu.async_copy(im_hbm.at[ix0], gt0, sem0)

    def body(k, carry):
        a = 2 * k
        b = a + 1
        compute(b, ix1, wt1)
        pltpu.make_async_copy(im_hbm.at[ix0], gt0, sem0).wait()
        combine_write(a, wt0, gt0, ob0, semw0, k == 0)
        pltpu.async_copy(im_hbm.at[ix1], gt1, sem1)

        @pl.when(k < NITER - 1)
        def _():
            compute(a + 2, ix0, wt0)

        pltpu.make_async_copy(im_hbm.at[ix1], gt1, sem1).wait()
        combine_write(b, wt1, gt1, ob1, semw1, k == 0)

        @pl.when(k < NITER - 1)
        def _():
            pltpu.async_copy(im_hbm.at[ix0], gt0, sem0)

        return carry

    lax.fori_loop(0, NITER, body, 0)
    pltpu.make_async_copy(im_hbm.at[pl.ds(0, CB)], ob0, semw0).wait()
    pltpu.make_async_copy(im_hbm.at[pl.ds(0, CB)], ob1, semw1).wait()


def _round_bf16(x):
    b = lax.bitcast_convert_type(x, jnp.uint32)
    b = (b + jnp.uint32(0x7FFF) + ((b >> 16) & jnp.uint32(1))) & jnp.uint32(
        0xFFFF0000)
    return lax.bitcast_convert_type(b, jnp.float32)


def kernel(stimuli, eye):
    im = stimuli.reshape(-1).astype(jnp.float32)
    ab = _round_bf16(eye.reshape(NF, 6).astype(jnp.float32))
    xt = jnp.linspace(-1.0, 1.0, W).astype(jnp.float32)
    yt = jnp.linspace(-1.0, 1.0, H).astype(jnp.float32)
    xg = _round_bf16(xt)
    yg = _round_bf16(yt)
    rx = ab[:, 1:2] * yg[None, :]
    ry = ab[:, 4:5] * yg[None, :]
    rx16 = jnp.broadcast_to(rx[:, :, None], (NF, H, L)).reshape(NF, H * L)
    ry16 = jnp.broadcast_to(ry[:, :, None], (NF, H, L)).reshape(NF, H * L)
    consts = jnp.stack([ab[:, 0], ab[:, 2], ab[:, 3], ab[:, 5]], axis=1)
    consts = jnp.pad(consts, ((0, 0), (0, 12)))
    out = _warp(im, consts, xg, rx16, ry16)
    return out.reshape(stimuli.shape)


# speedup vs baseline: 3.6992x; 3.6992x over previous
import functools

import jax
import jax.numpy as jnp
from jax import lax
from jax.experimental import pallas as pl
from jax.experimental.pallas import tpu as pltpu
from jax.experimental.pallas import tpu_sc as plsc

H, W = 304, 608
HW = H * W
PW = HW // 2
NF = 32
NC, NS, L = 2, 16, 16
ROWS_PER_CHUNK = 4
CB = ROWS_PER_CHUNK * W
NCHUNK = H // ROWS_PER_CHUNK
NITER = NCHUNK // 2
GPR = W // L
SB = 2 * CB
NBLK = HW // SB

_mesh = plsc.VectorSubcoreMesh(
    core_axis_name="c", subcore_axis_name="s", num_cores=NC, num_subcores=NS)

_MASK_HI = jnp.uint32(0xFFFF0000)


def _rne_hi(u):
    return (u + jnp.uint32(0x7FFF) + ((u >> 16) & jnp.uint32(1))) & _MASK_HI


@functools.partial(
    pl.kernel,
    out_type=jax.ShapeDtypeStruct((NF * HW,), jnp.float32),
    mesh=_mesh,
    compiler_params=pltpu.CompilerParams(needs_layout_passes=False),
    scratch_types=[
        pltpu.VMEM((16,), jnp.float32),
        pltpu.VMEM((W,), jnp.float32),
        pltpu.VMEM((H * L,), jnp.float32),
        pltpu.VMEM((H * L,), jnp.float32),
        pltpu.VMEM((PW,), jnp.int32),
        pltpu.VMEM((SB,), jnp.float32),
        pltpu.VMEM((CB,), jnp.float32),
        pltpu.VMEM((CB,), jnp.float32),
        pltpu.SemaphoreType.DMA,
        pltpu.SemaphoreType.DMA,
    ],
)
def _warp(im_hbm, consts_hbm, xg_hbm, rx_hbm, ry_hbm, out_hbm,
          cv, xgv, rxv, ryv, img, stg, ob0, ob1, semw0, semw1):
    wid = lax.axis_index("s") * NC + lax.axis_index("c")
    pltpu.sync_copy(consts_hbm.at[wid], cv)
    pltpu.sync_copy(xg_hbm, xgv)
    pltpu.sync_copy(rx_hbm.at[wid], rxv)
    pltpu.sync_copy(ry_hbm.at[wid], ryv)
    cvv = cv[...]
    a00 = jnp.full((L,), cvv[0])
    a02 = jnp.full((L,), cvv[1])
    a10 = jnp.full((L,), cvv[2])
    a12 = jnp.full((L,), cvv[3])
    base = wid * HW
    iota = lax.iota(jnp.int32, L)

    def pack_block(blk, carry):
        pltpu.sync_copy(im_hbm.at[pl.ds(base + blk * SB, SB)], stg)

        def grp(t, _):
            sl = 2 * iota + 32 * t
            ev = plsc.load_gather(stg, [sl])
            od = plsc.load_gather(stg, [sl + 1])
            eu = _rne_hi(plsc.bitcast(ev, jnp.uint32))
            ou = _rne_hi(plsc.bitcast(od, jnp.uint32))
            img[pl.ds(blk * (SB // 2) + t * L, L)] = plsc.bitcast(
                (eu >> 16) | ou, jnp.int32)
            return 0

        lax.fori_loop(0, SB // (2 * L), grp, 0)
        return carry

    lax.fori_loop(0, NBLK, pack_block, 0)

    def do_chunk(c, ob, semw, first):
        @pl.when(jnp.logical_not(first))
        def _():
            pltpu.make_async_copy(im_hbm.at[pl.ds(0, CB)], ob, semw).wait()

        def row_body(r, rc):
            i = c * ROWS_PER_CHUNK + r
            px = rxv[pl.ds(i * L, L)]
            py = ryv[pl.ds(i * L, L)]

            def grp_body(g, _):
                xv = xgv[pl.ds(g * L, L)]
                xs = (a00 * xv + px) + a02
                ys = (a10 * xv + py) + a12
                xp = (xs + 1.0) * (W / 2)
                yp = (ys + 1.0) * (H / 2)
                xq = jnp.clip(xp, -1e6, 1e6)
                yq = jnp.clip(yp, -1e6, 1e6)
                xt = xq.astype(jnp.int32)
                xtf = xt.astype(jnp.float32)
                neg_x = xq < xtf
                fxl = jnp.where(neg_x, xt - 1, xt)
                fxf = jnp.where(neg_x, xtf - 1.0, xtf)
                yt = yq.astype(jnp.int32)
                ytf = yt.astype(jnp.float32)
                neg_y = yq < ytf
                fyl = jnp.where(neg_y, yt - 1, yt)
                fyf = jnp.where(neg_y, ytf - 1.0, ytf)
                in_x = (xp >= 0.0) & (xp < W - 1.0)
                in_y = (yp >= 0.0) & (yp < H - 1.0)
                zero = jnp.zeros((L,), jnp.float32)
                hl = jnp.where(in_x, fxf + 1.0 - xp, zero)
                hr = jnp.where(in_x, xp - fxf, zero)
                vt = jnp.where(in_y, fyf + 1.0 - yp, zero)
                vb = jnp.where(in_y, yp - fyf, zero)
                x0 = jnp.clip(fxl, 0, W - 2)
                y0 = jnp.clip(fyl, 0, H - 2)
                lin = y0 * W + x0
                bit = lin & 1
                k0 = lin >> 1
                kb = k0 + (W // 2)
                wA = plsc.bitcast(plsc.load_gather(img, [k0]), jnp.uint32)
                wB = plsc.bitcast(plsc.load_gather(img, [k0 + bit]), jnp.uint32)
                wC = plsc.bitcast(plsc.load_gather(img, [kb]), jnp.uint32)
                wD = plsc.bitcast(plsc.load_gather(img, [kb + bit]), jnp.uint32)
                hi = bit == 1
                ia_ = jnp.where(hi, wA & _MASK_HI, wA << 16)
                ic_ = jnp.where(hi, wB << 16, wB & _MASK_HI)
                ib_ = jnp.where(hi, wC & _MASK_HI, wC << 16)
                id_ = jnp.where(hi, wD << 16, wD & _MASK_HI)
                fa = plsc.bitcast(ia_, jnp.float32)
                fc = plsc.bitcast(ic_, jnp.float32)
                fb = plsc.bitcast(ib_, jnp.float32)
                fd = plsc.bitcast(id_, jnp.float32)
                acc = ((hl * vt) * fa + (hl * vb) * fb
                       + (hr * vt) * fc + (hr * vb) * fd)
                ob[pl.ds((r * GPR + g) * L, L)] = acc
                return 0

            lax.fori_loop(0, GPR, grp_body, 0)
            return rc

        lax.fori_loop(0, ROWS_PER_CHUNK, row_body, 0)
        pltpu.async_copy(ob, out_hbm.at[pl.ds(base + c * CB, CB)], semw)

    def body(k, carry):
        do_chunk(2 * k, ob0, semw0, k == 0)
        do_chunk(2 * k + 1, ob1, semw1, k == 0)
        return carry

    lax.fori_loop(0, NITER, body, 0)
    pltpu.make_async_copy(im_hbm.at[pl.ds(0, CB)], ob0, semw0).wait()
    pltpu.make_async_copy(im_hbm.at[pl.ds(0, CB)], ob1, semw1).wait()


def _round_bf16(x):
    b = lax.bitcast_convert_type(x, jnp.uint32)
    b = (b + jnp.uint32(0x7FFF) + ((b >> 16) & jnp.uint32(1))) & jnp.uint32(
        0xFFFF0000)
    return lax.bitcast_convert_type(b, jnp.float32)


def kernel(stimuli, eye):
    im = stimuli.reshape(-1).astype(jnp.float32)
    ab = _round_bf16(eye.reshape(NF, 6).astype(jnp.float32))
    xt = jnp.linspace(-1.0, 1.0, W).astype(jnp.float32)
    yt = jnp.linspace(-1.0, 1.0, H).astype(jnp.float32)
    xg = _round_bf16(xt)
    yg = _round_bf16(yt)
    rx = ab[:, 1:2] * yg[None, :]
    ry = ab[:, 4:5] * yg[None, :]
    rx16 = jnp.broadcast_to(rx[:, :, None], (NF, H, L)).reshape(NF, H * L)
    ry16 = jnp.broadcast_to(ry[:, :, None], (NF, H, L)).reshape(NF, H * L)
    consts = jnp.stack([ab[:, 0], ab[:, 2], ab[:, 3], ab[:, 5]], axis=1)
    consts = jnp.pad(consts, ((0, 0), (0, 12)))
    out = _warp(im, consts, xg, rx16, ry16)
    return out.reshape(stimuli.shape)
